# baseline (speedup 1.0000x reference)
import functools

import jax
import jax.numpy as jnp
from jax import lax
from jax.experimental import pallas as pl
from jax.experimental.pallas import tpu as pltpu
from jax.experimental.pallas import tpu_sc as plsc

B = 4096
NT = 16
CHUNK = B // NT
NV = CHUNK // 16
NROW = 24

_PAIRS = ((0, 1), (0, 2), (0, 3), (1, 2), (1, 3), (2, 3))


def _recon(m):
    M, ML, MG, MH, MLG, MLH, MGH, MLGH = m
    return [
        M - ML - MG - MH + MLG + MLH + MGH - MLGH,
        MH - MLH - MGH + MLGH,
        MG - MGH - MLG + MLGH,
        MGH - MLGH,
        ML - MLG - MLH + MLGH,
        MLH - MLGH,
        MLG - MLGH,
        MLGH,
    ]


def _body(out_hbm, lab_hbm, sen_hbm, ad1_hbm, ad2_hbm, amap_hbm,
          res_hbm, stage_hbm,
          x_v, lab_v, g1_v, g2_v, ad1_v, ad2_v, amap_v,
          acc_v, all_v, res_v, sem):
    c = lax.axis_index("c")
    s = lax.axis_index("s")

    @pl.when(c == 0)
    def _compute_partials():
        base = s * CHUNK
        copies = [
            pltpu.async_copy(out_hbm.at[pl.ds(base, CHUNK)], x_v, sem),
            pltpu.async_copy(lab_hbm.at[pl.ds(base, CHUNK)], lab_v, sem),
            pltpu.async_copy(sen_hbm.at[0, pl.ds(base, CHUNK)], g1_v, sem),
            pltpu.async_copy(sen_hbm.at[1, pl.ds(base, CHUNK)], g2_v, sem),
            pltpu.async_copy(ad1_hbm.at[pl.ds(base, CHUNK)], ad1_v, sem),
            pltpu.async_copy(ad2_hbm.at[pl.ds(base, CHUNK)], ad2_v, sem),
            pltpu.async_copy(amap_hbm.at[0], amap_v, sem),
        ]
        for cp in copies:
            cp.wait()

        zero16 = jnp.zeros((16,), jnp.float32)
        ones16 = jnp.ones((16,), jnp.float32)
        zero16i = jnp.zeros((16,), jnp.int32)
        a0 = plsc.load_gather(amap_v, [zero16i])
        a1 = plsc.load_gather(amap_v, [zero16i + 1])

        accs = [zero16] * 17
        for j in range(NV):
            dsl = pl.ds(j * 16, 16)
            x = x_v[dsl]
            u = jnp.exp(-jnp.abs(x))
            z = u / (u + 2.0)
            z2 = z * z
            p = z2 * (1.0 / 9.0) + (1.0 / 7.0)
            p = p * z2 + (1.0 / 5.0)
            p = p * z2 + (1.0 / 3.0)
            p = p * z2 + 1.0
            ls = jnp.minimum(x, 0.0) - 2.0 * z * p
            dom = (ad1_v[dsl] == a0) & (ad2_v[dsl] == a1)
            domf = jnp.where(dom, ones16, zero16)
            L = lab_v[dsl].astype(jnp.float32)
            G = g1_v[dsl].astype(jnp.float32)
            H = g2_v[dsl].astype(jnp.float32)
            dL = domf * L
            dG = domf * G
            dH = domf * H
            dLG = dL * G
            dLH = dL * H
            dGH = dG * H
            dLGH = dLG * H
            terms = (domf, dL, dG, dH, dLG, dLH, dGH, dLGH)
            for k in range(8):
                accs[k] = accs[k] + ls * terms[k]
                accs[k + 8] = accs[k + 8] + terms[k]
            accs[16] = accs[16] + L
        for r in range(17):
            acc_v[r, :] = accs[r]
        for r in range(17, NROW):
            acc_v[r, :] = zero16
        pltpu.sync_copy(acc_v, stage_hbm.at[s])

    plsc.subcore_barrier()

    @pl.when((c == 0) & (s == 0))
    def _finalize():
        gathers = [
            pltpu.async_copy(stage_hbm.at[t],
                             all_v.at[pl.ds(t * NROW, NROW)], sem)
            for t in range(NT)
        ]
        for g in gathers:
            g.wait()
        zero16 = jnp.zeros((16,), jnp.float32)
        ones16 = jnp.ones((16,), jnp.float32)
        totals = []
        for r in range(17):
            acc = all_v[r, :]
            for t in range(1, NT):
                acc = acc + all_v[t * NROW + r, :]
            totals.append(jnp.full((16,), jnp.sum(acc)))
        sums = _recon(totals[0:8])
        cnts = _recon(totals[8:16])
        means = [sums[b] / jnp.maximum(cnts[b], ones16) for b in range(8)]
        pres = [jnp.where(cnts[b] > 0.0, ones16, zero16) for b in range(8)]
        labtot = totals[16]
        has = [jnp.where(labtot < float(B), ones16, zero16),
               jnp.where(labtot > 0.0, ones16, zero16)]
        res = zero16
        for l in range(2):
            gap = zero16
            for (i, j) in _PAIRS:
                w = pres[4 * l + i] * pres[4 * l + j]
                d = means[4 * l + i] - means[4 * l + j]
                gap = gap + w * d * d
            res = res + has[l] * gap
        res_v[:] = res
        pltpu.sync_copy(res_v.at[pl.ds(0, 1)], res_hbm)


@jax.jit
def _mean_loss_sc(outputs, labels, sen_groups, ad1, ad2, a_map):
    kfn = pl.kernel(
        _body,
        out_type=(jax.ShapeDtypeStruct((1,), jnp.float32),
                  jax.ShapeDtypeStruct((NT, NROW, 16), jnp.float32)),
        mesh=plsc.VectorSubcoreMesh(core_axis_name="c", subcore_axis_name="s"),
        compiler_params=pltpu.CompilerParams(needs_layout_passes=False),
        scratch_types=[
            pltpu.VMEM((CHUNK,), jnp.float32),
            pltpu.VMEM((CHUNK,), jnp.int32),
            pltpu.VMEM((CHUNK,), jnp.int32),
            pltpu.VMEM((CHUNK,), jnp.int32),
            pltpu.VMEM((CHUNK,), jnp.int32),
            pltpu.VMEM((CHUNK,), jnp.int32),
            pltpu.VMEM((2,), jnp.int32),
            pltpu.VMEM((NROW, 16), jnp.float32),
            pltpu.VMEM((NT * NROW, 16), jnp.float32),
            pltpu.VMEM((16,), jnp.float32),
            pltpu.SemaphoreType.DMA,
        ],
    )
    res, _stage = kfn(outputs, labels, sen_groups, ad1, ad2, a_map)
    return res


def kernel(outputs, labels, sen_group_name, sen_groups, ad1, ad2, a_map):
    return _mean_loss_sc(outputs, labels, sen_groups, ad1, ad2, a_map)


# speedup vs baseline: 1.1040x; 1.1040x over previous
import functools

import jax
import jax.numpy as jnp
from jax import lax
from jax.experimental import pallas as pl
from jax.experimental.pallas import tpu as pltpu
from jax.experimental.pallas import tpu_sc as plsc

B = 4096
NT = 16
CHUNK = B // NT
NV = CHUNK // 16
NROW = 8

_PAIRS = ((0, 1), (0, 2), (0, 3), (1, 2), (1, 3), (2, 3))


def _recon(m):
    M, ML, MG, MH, MLG, MLH, MGH, MLGH = m
    return [
        M - ML - MG - MH + MLG + MLH + MGH - MLGH,
        MH - MLH - MGH + MLGH,
        MG - MGH - MLG + MLGH,
        MGH - MLGH,
        ML - MLG - MLH + MLGH,
        MLH - MLGH,
        MLG - MLGH,
        MLGH,
    ]


def _body(out_hbm, lab_hbm, sen_hbm, ad1_hbm, ad2_hbm, amap_hbm,
          res_hbm, stage_hbm,
          x_v, lab_v, g1_v, g2_v, ad1_v, ad2_v, amap_v,
          acc_v, all_v, res_v, sem):
    c = lax.axis_index("c")
    s = lax.axis_index("s")

    @pl.when(c == 0)
    def _compute_partials():
        base = s * CHUNK
        copies = [
            pltpu.async_copy(out_hbm.at[pl.ds(base, CHUNK)], x_v, sem),
            pltpu.async_copy(lab_hbm.at[pl.ds(base, CHUNK)], lab_v, sem),
            pltpu.async_copy(sen_hbm.at[0, pl.ds(base, CHUNK)], g1_v, sem),
            pltpu.async_copy(sen_hbm.at[1, pl.ds(base, CHUNK)], g2_v, sem),
            pltpu.async_copy(ad1_hbm.at[pl.ds(base, CHUNK)], ad1_v, sem),
            pltpu.async_copy(ad2_hbm.at[pl.ds(base, CHUNK)], ad2_v, sem),
            pltpu.async_copy(amap_hbm.at[0], amap_v, sem),
        ]
        for cp in copies:
            cp.wait()

        zero16 = jnp.zeros((16,), jnp.float32)
        ones16 = jnp.ones((16,), jnp.float32)
        zero16i = jnp.zeros((16,), jnp.int32)
        lanes = lax.iota(jnp.int32, 16)
        a0 = plsc.load_gather(amap_v, [zero16i])
        a1 = plsc.load_gather(amap_v, [zero16i + 1])

        accs = [zero16] * 17
        for j in range(NV):
            dsl = pl.ds(j * 16, 16)
            x = x_v[dsl]
            u = jnp.exp(-jnp.abs(x))
            z = u / (u + 2.0)
            z2 = z * z
            p = z2 * (1.0 / 9.0) + (1.0 / 7.0)
            p = p * z2 + (1.0 / 5.0)
            p = p * z2 + (1.0 / 3.0)
            p = p * z2 + 1.0
            ls = jnp.minimum(x, 0.0) - 2.0 * z * p
            dom = (ad1_v[dsl] == a0) & (ad2_v[dsl] == a1)
            domf = jnp.where(dom, ones16, zero16)
            L = lab_v[dsl].astype(jnp.float32)
            G = g1_v[dsl].astype(jnp.float32)
            H = g2_v[dsl].astype(jnp.float32)
            dL = domf * L
            dG = domf * G
            dH = domf * H
            dLG = dL * G
            dLH = dL * H
            dGH = dG * H
            dLGH = dLG * H
            terms = (domf, dL, dG, dH, dLG, dLH, dGH, dLGH)
            for k in range(8):
                accs[k] = accs[k] + ls * terms[k]
                accs[k + 8] = accs[k + 8] + terms[k]
            accs[16] = accs[16] + L
        row0 = zero16
        for r in range(16):
            row0 = row0 + jnp.where(lanes == r,
                                    jnp.full((16,), jnp.sum(accs[r])), zero16)
        row1 = jnp.where(lanes == 0,
                         jnp.full((16,), jnp.sum(accs[16])), zero16)
        acc_v[0, :] = row0
        acc_v[1, :] = row1
        for r in range(2, NROW):
            acc_v[r, :] = zero16
        pltpu.sync_copy(acc_v, stage_hbm.at[pl.ds(s * NROW, NROW)])

    plsc.subcore_barrier()

    @pl.when((c == 0) & (s == 0))
    def _finalize():
        pltpu.sync_copy(stage_hbm, all_v)
        zero16 = jnp.zeros((16,), jnp.float32)
        ones16 = jnp.ones((16,), jnp.float32)
        lanes = lax.iota(jnp.int32, 16)
        mom = all_v[0, :]
        labv = all_v[1, :]
        for t in range(1, NT):
            mom = mom + all_v[t * NROW, :]
            labv = labv + all_v[t * NROW + 1, :]
        totals = [jnp.full((16,), jnp.sum(jnp.where(lanes == r, mom, zero16)))
                  for r in range(16)]
        labtot = jnp.full((16,), jnp.sum(labv))
        sums = _recon(totals[0:8])
        cnts = _recon(totals[8:16])
        means = [sums[b] / jnp.maximum(cnts[b], ones16) for b in range(8)]
        pres = [jnp.where(cnts[b] > 0.0, ones16, zero16) for b in range(8)]
        has = [jnp.where(labtot < float(B), ones16, zero16),
               jnp.where(labtot > 0.0, ones16, zero16)]
        res = zero16
        for l in range(2):
            gap = zero16
            for (i, j) in _PAIRS:
                w = pres[4 * l + i] * pres[4 * l + j]
                d = means[4 * l + i] - means[4 * l + j]
                gap = gap + w * d * d
            res = res + has[l] * gap
        res_v[:] = res
        pltpu.sync_copy(res_v.at[pl.ds(0, 1)], res_hbm)


@jax.jit
def _mean_loss_sc(outputs, labels, sen_groups, ad1, ad2, a_map):
    kfn = pl.kernel(
        _body,
        out_type=(jax.ShapeDtypeStruct((1,), jnp.float32),
                  jax.ShapeDtypeStruct((NT * NROW, 16), jnp.float32)),
        mesh=plsc.VectorSubcoreMesh(core_axis_name="c", subcore_axis_name="s"),
        compiler_params=pltpu.CompilerParams(needs_layout_passes=False),
        scratch_types=[
            pltpu.VMEM((CHUNK,), jnp.float32),
            pltpu.VMEM((CHUNK,), jnp.int32),
            pltpu.VMEM((CHUNK,), jnp.int32),
            pltpu.VMEM((CHUNK,), jnp.int32),
            pltpu.VMEM((CHUNK,), jnp.int32),
            pltpu.VMEM((CHUNK,), jnp.int32),
            pltpu.VMEM((2,), jnp.int32),
            pltpu.VMEM((NROW, 16), jnp.float32),
            pltpu.VMEM((NT * NROW, 16), jnp.float32),
            pltpu.VMEM((16,), jnp.float32),
            pltpu.SemaphoreType.DMA,
        ],
    )
    res, _stage = kfn(outputs, labels, sen_groups, ad1, ad2, a_map)
    return res


def kernel(outputs, labels, sen_group_name, sen_groups, ad1, ad2, a_map):
    return _mean_loss_sc(outputs, labels, sen_groups, ad1, ad2, a_map)
